# speedup vs baseline: 1.1017x; 1.1017x over previous
import functools

import jax
import jax.numpy as jnp
from jax import lax
from jax.experimental import pallas as pl
from jax.experimental.pallas import tpu as pltpu
from jax.experimental.pallas import tpu_sc as plsc

_CHUNK = 128


@functools.cache
def _build(batch: int, n_states: int):
  info = plsc.get_sparse_core_info()
  nw = info.num_cores * info.num_subcores
  rows = batch // _CHUNK
  rows_per_w = rows // nw

  mesh = plsc.VectorSubcoreMesh(core_axis_name="c", subcore_axis_name="s")

  @functools.partial(
      pl.kernel,
      mesh=mesh,
      out_type=jax.ShapeDtypeStruct((rows, _CHUNK), jnp.float32),
      scratch_types=[
          pltpu.VMEM((rows_per_w, _CHUNK), jnp.int32),
          pltpu.VMEM((rows_per_w, _CHUNK), jnp.float32),
          pltpu.SemaphoreType.DMA,
          pltpu.SemaphoreType.DMA,
      ],
  )
  def gather_kernel(state_hbm, value_hbm, out_hbm, idx_v, vals_v, sem_i, sem_g):
    wid = lax.axis_index("s") * info.num_cores + lax.axis_index("c")
    base = wid * rows_per_w
    pltpu.async_copy(state_hbm.at[pl.ds(base, rows_per_w)], idx_v, sem_i).wait()
    copies = []
    for j in range(rows_per_w):
      copies.append(
          pltpu.async_copy(value_hbm.at[idx_v.at[j]], vals_v.at[j], sem_g))
    for c in copies:
      c.wait()
    pltpu.sync_copy(vals_v, out_hbm.at[pl.ds(base, rows_per_w)])

  return gather_kernel


def kernel(state, value):
  batch = state.shape[0]
  state2 = state.astype(jnp.int32).reshape(batch // _CHUNK, _CHUNK)
  out = _build(batch, value.shape[0])(state2, value)
  return out.reshape(batch)


# speedup vs baseline: 1.1142x; 1.0113x over previous
import functools

import jax
import jax.numpy as jnp
from jax import lax
from jax.experimental import pallas as pl
from jax.experimental.pallas import tpu as pltpu
from jax.experimental.pallas import tpu_sc as plsc

_CHUNK = 128


@functools.cache
def _build(batch: int, n_states: int):
  info = plsc.get_sparse_core_info()
  nw = info.num_cores * info.num_subcores
  rows = batch // _CHUNK
  rows_per_w = rows // nw

  mesh = plsc.VectorSubcoreMesh(core_axis_name="c", subcore_axis_name="s")

  @functools.partial(
      pl.kernel,
      mesh=mesh,
      out_type=jax.ShapeDtypeStruct((rows * _CHUNK,), jnp.float32),
      scratch_types=[
          pltpu.VMEM((rows_per_w * _CHUNK,), jnp.int32),
          pltpu.VMEM((rows_per_w * _CHUNK,), jnp.float32),
          pltpu.SemaphoreType.DMA,
          pltpu.SemaphoreType.DMA,
      ],
  )
  def gather_kernel(state_hbm, value_hbm, out_hbm, idx_v, vals_v, sem_i, sem_g):
    wid = lax.axis_index("s") * info.num_cores + lax.axis_index("c")
    n_per_w = rows_per_w * _CHUNK
    base = wid * n_per_w
    pltpu.async_copy(state_hbm.at[pl.ds(base, n_per_w)], idx_v, sem_i).wait()
    pltpu.async_copy(value_hbm.at[idx_v], vals_v, sem_g).wait()
    pltpu.sync_copy(vals_v, out_hbm.at[pl.ds(base, n_per_w)])

  return gather_kernel


def kernel(state, value):
  batch = state.shape[0]
  return _build(batch, value.shape[0])(state.astype(jnp.int32), value)
